# Initial kernel scaffold; baseline (speedup 1.0000x reference)
import functools

import jax
import jax.numpy as jnp
from jax import lax
from jax.experimental import pallas as pl
from jax.experimental.pallas import tpu as pltpu
from jax.experimental.pallas import tpu_sc as plsc

_N, _E, _D, _H, _OUT, _G = 10000, 320000, 128, 128, 12, 256
_NC, _NS = 2, 16
_NW = _NC * _NS
_EPW = _E // _NW
_CH = 80
_NCH = _EPW // _CH
_RPT = _N // _NS

_sc_mesh = plsc.VectorSubcoreMesh(
    core_axis_name="c", subcore_axis_name="s", num_cores=_NC, num_subcores=_NS)


@functools.partial(
    pl.kernel,
    out_type=jax.ShapeDtypeStruct((_NC, _N, _D), jnp.float32),
    mesh=_sc_mesh,
    scratch_types=[
        pltpu.VMEM_SHARED((_N, _D), jnp.float32),
        pltpu.VMEM((_RPT, _D), jnp.float32),
        pltpu.VMEM((_CH,), jnp.int32),
        pltpu.VMEM((_CH,), jnp.int32),
        pltpu.VMEM((_CH, _D), jnp.float32),
        pltpu.SemaphoreType.DMA,
    ],
)
def _sc_agg(x_hbm, src_hbm, dst_hbm, z_hbm, out_hbm,
            acc_sh, zb_v, src_v, dst_v, rows_v, sem):
    c = lax.axis_index("c")
    s = lax.axis_index("s")
    pltpu.sync_copy(z_hbm, zb_v)
    pltpu.sync_copy(zb_v, acc_sh.at[pl.ds(s * _RPT, _RPT)])
    plsc.subcore_barrier()

    ebase = c * (_E // _NC) + s * _EPW

    def chunk(i, carry):
        b = ebase + i * _CH
        pltpu.sync_copy(src_hbm.at[pl.ds(b, _CH)], src_v)
        pltpu.sync_copy(dst_hbm.at[pl.ds(b, _CH)], dst_v)
        pltpu.async_copy(x_hbm.at[src_v], rows_v, sem).wait()
        pltpu.sync_copy(rows_v, acc_sh.at[dst_v], add=True)
        return carry

    lax.fori_loop(0, _NCH, chunk, 0)
    plsc.subcore_barrier()
    pltpu.sync_copy(acc_sh.at[pl.ds(s * _RPT, _RPT)],
                    out_hbm.at[c, pl.ds(s * _RPT, _RPT)])


def _dense_body(x_ref, p_ref, w1_ref, b1_ref, w2_ref, b2_ref, g_ref, bt_ref,
                o_ref):
    h = x_ref[...] + p_ref[0] + p_ref[1]
    h = jnp.maximum(
        jnp.dot(h, w1_ref[...], preferred_element_type=jnp.float32)
        + b1_ref[...], 0.0)
    h = jnp.dot(h, w2_ref[...], preferred_element_type=jnp.float32) + b2_ref[...]
    mu = jnp.mean(h, axis=0, keepdims=True)
    var = jnp.mean(jnp.square(h - mu), axis=0, keepdims=True)
    h = (h - mu) * lax.rsqrt(var + 1e-5) * g_ref[...] + bt_ref[...]
    o_ref[...] = jnp.maximum(h, 0.0)


_dense = pl.pallas_call(
    _dense_body,
    out_shape=jax.ShapeDtypeStruct((_N, _H), jnp.float32),
)


def _final_body(x_ref, p_ref, batch_ref, w1_ref, b1_ref, w2_ref, b2_ref,
                g_ref, bt_ref, wh1_ref, bh1_ref, wh2_ref, bh2_ref, o_ref):
    h = x_ref[...] + p_ref[0] + p_ref[1]
    h = jnp.maximum(
        jnp.dot(h, w1_ref[...], preferred_element_type=jnp.float32)
        + b1_ref[...], 0.0)
    h = jnp.dot(h, w2_ref[...], preferred_element_type=jnp.float32) + b2_ref[...]
    mu = jnp.mean(h, axis=0, keepdims=True)
    var = jnp.mean(jnp.square(h - mu), axis=0, keepdims=True)
    h = (h - mu) * lax.rsqrt(var + 1e-5) * g_ref[...] + bt_ref[...]
    h = jnp.maximum(h, 0.0)
    gids = lax.broadcasted_iota(jnp.int32, (_G, _N), 0)
    onehot = (batch_ref[...] == gids).astype(jnp.float32)
    pool = jnp.dot(onehot, h, preferred_element_type=jnp.float32)
    q = jnp.maximum(
        jnp.dot(pool, wh1_ref[...], preferred_element_type=jnp.float32)
        + bh1_ref[...], 0.0)
    o_ref[...] = jnp.dot(q, wh2_ref[...],
                         preferred_element_type=jnp.float32) + bh2_ref[...]


_final = pl.pallas_call(
    _final_body,
    out_shape=jax.ShapeDtypeStruct((_G, _OUT), jnp.float32),
)


def kernel(x, edge_index, batch, W1_0, b1_0, W2_0, b2_0, g_0, bt_0,
           W1_1, b1_1, W2_1, b2_1, g_1, bt_1,
           W1_2, b1_2, W2_2, b2_2, g_2, bt_2, Wh1, bh1, Wh2, bh2):
    src = edge_index[0]
    dst = edge_index[1]
    zrows = jnp.zeros((_RPT, _D), jnp.float32)
    r2 = lambda v: v.reshape(1, -1)

    p = _sc_agg(x, src, dst, zrows)
    h = _dense(x, p, W1_0, r2(b1_0), W2_0, r2(b2_0), r2(g_0), r2(bt_0))
    p = _sc_agg(h, src, dst, zrows)
    h = _dense(h, p, W1_1, r2(b1_1), W2_1, r2(b2_1), r2(g_1), r2(bt_1))
    p = _sc_agg(h, src, dst, zrows)
    return _final(h, p, batch.reshape(1, -1), W1_2, r2(b1_2), W2_2, r2(b2_2),
                  r2(g_2), r2(bt_2), Wh1, r2(bh1), Wh2, r2(bh2))


# speedup vs baseline: 2.8701x; 2.8701x over previous
import functools

import jax
import jax.numpy as jnp
from jax import lax
from jax.experimental import pallas as pl
from jax.experimental.pallas import tpu as pltpu
from jax.experimental.pallas import tpu_sc as plsc

_N, _E, _D, _H, _OUT, _G = 10000, 320000, 128, 128, 12, 256
_HD = _D // 2
_NC, _NS = 2, 16
_EPT = _E // _NS
_CH = 80
_NCH = _EPT // _CH
_NP = 10240
_RPT = _NP // _NS

_sc_mesh = plsc.VectorSubcoreMesh(
    core_axis_name="c", subcore_axis_name="s", num_cores=_NC, num_subcores=_NS)


@functools.partial(
    pl.kernel,
    out_type=jax.ShapeDtypeStruct((_NC, _NP, _HD), jnp.float32),
    mesh=_sc_mesh,
    scratch_types=[
        pltpu.VMEM_SHARED((_NP, _HD), jnp.float32),
        pltpu.VMEM((_RPT, _HD), jnp.float32),
        pltpu.VMEM((_CH,), jnp.int32),
        pltpu.VMEM((_CH,), jnp.int32),
        pltpu.VMEM((_CH, _HD), jnp.float32),
        pltpu.SemaphoreType.DMA,
    ],
    compiler_params=pltpu.CompilerParams(use_tc_tiling_on_sc=False),
)
def _sc_agg(x0_hbm, x1_hbm, src_hbm, dst_hbm, z_hbm, out_hbm,
            acc_sh, zb_v, src_v, dst_v, rows_v, sem):
    c = lax.axis_index("c")
    s = lax.axis_index("s")
    pltpu.sync_copy(z_hbm, zb_v)
    pltpu.sync_copy(zb_v, acc_sh.at[pl.ds(s * _RPT, _RPT)])
    plsc.subcore_barrier()

    ebase = s * _EPT

    def edge_loop(x_hbm):
        def chunk(i, carry):
            b = ebase + i * _CH
            pltpu.sync_copy(src_hbm.at[pl.ds(b, _CH)], src_v)
            pltpu.sync_copy(dst_hbm.at[pl.ds(b, _CH)], dst_v)
            pltpu.async_copy(x_hbm.at[src_v], rows_v, sem).wait()
            pltpu.sync_copy(rows_v, acc_sh.at[dst_v], add=True)
            return carry
        lax.fori_loop(0, _NCH, chunk, 0)

    @pl.when(c == 0)
    def _():
        edge_loop(x0_hbm)

    @pl.when(c == 1)
    def _():
        edge_loop(x1_hbm)

    plsc.subcore_barrier()
    pltpu.sync_copy(acc_sh.at[pl.ds(s * _RPT, _RPT)],
                    out_hbm.at[c, pl.ds(s * _RPT, _RPT)])


def _mlp_bn(a, b, w1_ref, b1_ref, w2_ref, b2_ref, g_ref, bt_ref):
    h = jnp.dot(a, w1_ref[:_HD], preferred_element_type=jnp.float32)
    h += jnp.dot(b, w1_ref[_HD:], preferred_element_type=jnp.float32)
    h = jnp.maximum(h + b1_ref[...], 0.0)
    h = jnp.dot(h, w2_ref[...], preferred_element_type=jnp.float32) + b2_ref[...]
    mu = jnp.mean(h, axis=0, keepdims=True)
    var = jnp.mean(jnp.square(h - mu), axis=0, keepdims=True)
    h = (h - mu) * lax.rsqrt(var + 1e-5) * g_ref[...] + bt_ref[...]
    return jnp.maximum(h, 0.0)


def _dense_body(xl_ref, xh_ref, p_ref, w1_ref, b1_ref, w2_ref, b2_ref,
                g_ref, bt_ref, ol_ref, oh_ref):
    a = xl_ref[...] + p_ref[0, :_N]
    b = xh_ref[...] + p_ref[1, :_N]
    h = _mlp_bn(a, b, w1_ref, b1_ref, w2_ref, b2_ref, g_ref, bt_ref)
    ol_ref[...] = h[:, :_HD]
    oh_ref[...] = h[:, _HD:]


_dense = pl.pallas_call(
    _dense_body,
    out_shape=[jax.ShapeDtypeStruct((_N, _HD), jnp.float32),
               jax.ShapeDtypeStruct((_N, _HD), jnp.float32)],
)


def _final_body(xl_ref, xh_ref, p_ref, batch_ref, w1_ref, b1_ref, w2_ref,
                b2_ref, g_ref, bt_ref, wh1_ref, bh1_ref, wh2_ref, bh2_ref,
                o_ref):
    a = xl_ref[...] + p_ref[0, :_N]
    b = xh_ref[...] + p_ref[1, :_N]
    h = _mlp_bn(a, b, w1_ref, b1_ref, w2_ref, b2_ref, g_ref, bt_ref)
    gids = lax.broadcasted_iota(jnp.int32, (_G, _N), 0)
    onehot = (batch_ref[...] == gids).astype(jnp.float32)
    pool = jnp.dot(onehot, h, preferred_element_type=jnp.float32)
    q = jnp.maximum(
        jnp.dot(pool, wh1_ref[...], preferred_element_type=jnp.float32)
        + bh1_ref[...], 0.0)
    o_ref[...] = jnp.dot(q, wh2_ref[...],
                         preferred_element_type=jnp.float32) + bh2_ref[...]


_final = pl.pallas_call(
    _final_body,
    out_shape=jax.ShapeDtypeStruct((_G, _OUT), jnp.float32),
)


def kernel(x, edge_index, batch, W1_0, b1_0, W2_0, b2_0, g_0, bt_0,
           W1_1, b1_1, W2_1, b2_1, g_1, bt_1,
           W1_2, b1_2, W2_2, b2_2, g_2, bt_2, Wh1, bh1, Wh2, bh2):
    src = edge_index[0]
    dst = edge_index[1]
    zrows = jnp.zeros((_RPT, _HD), jnp.float32)
    r2 = lambda v: v.reshape(1, -1)
    hl, hh = x[:, :_HD], x[:, _HD:]

    p = _sc_agg(hl, hh, src, dst, zrows)
    hl, hh = _dense(hl, hh, p, W1_0, r2(b1_0), W2_0, r2(b2_0), r2(g_0),
                    r2(bt_0))
    p = _sc_agg(hl, hh, src, dst, zrows)
    hl, hh = _dense(hl, hh, p, W1_1, r2(b1_1), W2_1, r2(b2_1), r2(g_1),
                    r2(bt_1))
    p = _sc_agg(hl, hh, src, dst, zrows)
    return _final(hl, hh, p, batch.reshape(1, -1), W1_2, r2(b1_2), W2_2,
                  r2(b2_2), r2(g_2), r2(bt_2), Wh1, r2(bh1), Wh2, r2(bh2))


# speedup vs baseline: 4.5702x; 1.5924x over previous
import functools

import jax
import jax.numpy as jnp
from jax import lax
from jax.experimental import pallas as pl
from jax.experimental.pallas import tpu as pltpu
from jax.experimental.pallas import tpu_sc as plsc

_N, _E, _D, _H, _OUT, _G = 10000, 320000, 128, 128, 12, 256
_HD = _D // 2
_NC, _NS = 2, 16
_CH = 128
_NCH = 160
_EPT = _NCH * _CH
_EPAD = _NS * _EPT
_NB = 4
_NP = 10240
_RPT = _NP // _NS

_sc_mesh = plsc.VectorSubcoreMesh(
    core_axis_name="c", subcore_axis_name="s", num_cores=_NC, num_subcores=_NS)


@functools.partial(
    pl.kernel,
    out_type=jax.ShapeDtypeStruct((_NC, _NP, _HD), jnp.float32),
    mesh=_sc_mesh,
    scratch_types=[
        pltpu.VMEM_SHARED((_NP, _HD), jnp.float32),
        pltpu.VMEM((_NCH, _CH), jnp.int32),
        pltpu.VMEM((_NCH, _CH), jnp.int32),
        [pltpu.VMEM((_CH, _HD), jnp.float32)] * _NB,
        [pltpu.SemaphoreType.DMA] * _NB,
    ],
    compiler_params=pltpu.CompilerParams(use_tc_tiling_on_sc=False),
)
def _sc_agg(x0_hbm, x1_hbm, src_hbm, dst_hbm, z_hbm, out_hbm,
            acc_sh, src_v, dst_v, rows, sems):
    c = lax.axis_index("c")
    s = lax.axis_index("s")
    pltpu.sync_copy(z_hbm, rows[0])
    for k in range(_RPT // _CH):
        pltpu.sync_copy(rows[0], acc_sh.at[pl.ds(s * _RPT + k * _CH, _CH)])
    pltpu.sync_copy(src_hbm.at[s], src_v)
    pltpu.sync_copy(dst_hbm.at[s], dst_v)
    plsc.subcore_barrier()

    def edge_loop(x_hbm):
        for b in range(_NB):
            pltpu.async_copy(x_hbm.at[src_v.at[b]], rows[b], sems[b])

        @pl.loop(0, _NCH, step=_NB)
        def _(i0):
            for b in range(_NB):
                i = i0 + b
                pltpu.make_async_copy(x_hbm.at[src_v.at[i]], rows[b],
                                      sems[b]).wait()
                pltpu.sync_copy(rows[b], acc_sh.at[dst_v.at[i]], add=True)

                @pl.when(i + _NB < _NCH)
                def _():
                    pltpu.async_copy(x_hbm.at[src_v.at[i + _NB]], rows[b],
                                     sems[b])

    @pl.when(c == 0)
    def _():
        edge_loop(x0_hbm)

    @pl.when(c == 1)
    def _():
        edge_loop(x1_hbm)

    plsc.subcore_barrier()
    pltpu.sync_copy(acc_sh.at[pl.ds(s * _RPT, _RPT)],
                    out_hbm.at[c, pl.ds(s * _RPT, _RPT)])


def _mlp_bn(a, b, w1_ref, b1_ref, w2_ref, b2_ref, g_ref, bt_ref):
    h = jnp.dot(a, w1_ref[:_HD], preferred_element_type=jnp.float32)
    h += jnp.dot(b, w1_ref[_HD:], preferred_element_type=jnp.float32)
    h = jnp.maximum(h + b1_ref[...], 0.0)
    h = jnp.dot(h, w2_ref[...], preferred_element_type=jnp.float32) + b2_ref[...]
    mu = jnp.mean(h, axis=0, keepdims=True)
    var = jnp.mean(jnp.square(h - mu), axis=0, keepdims=True)
    h = (h - mu) * lax.rsqrt(var + 1e-5) * g_ref[...] + bt_ref[...]
    return jnp.maximum(h, 0.0)


def _dense_body(xl_ref, xh_ref, p_ref, w1_ref, b1_ref, w2_ref, b2_ref,
                g_ref, bt_ref, ol_ref, oh_ref):
    a = xl_ref[...] + p_ref[0, :_N]
    b = xh_ref[...] + p_ref[1, :_N]
    h = _mlp_bn(a, b, w1_ref, b1_ref, w2_ref, b2_ref, g_ref, bt_ref)
    ol_ref[...] = h[:, :_HD]
    oh_ref[...] = h[:, _HD:]


_dense = pl.pallas_call(
    _dense_body,
    out_shape=[jax.ShapeDtypeStruct((_N, _HD), jnp.float32),
               jax.ShapeDtypeStruct((_N, _HD), jnp.float32)],
)


def _final_body(xl_ref, xh_ref, p_ref, batch_ref, w1_ref, b1_ref, w2_ref,
                b2_ref, g_ref, bt_ref, wh1_ref, bh1_ref, wh2_ref, bh2_ref,
                o_ref):
    a = xl_ref[...] + p_ref[0, :_N]
    b = xh_ref[...] + p_ref[1, :_N]
    h = _mlp_bn(a, b, w1_ref, b1_ref, w2_ref, b2_ref, g_ref, bt_ref)
    gids = lax.broadcasted_iota(jnp.int32, (_G, _N), 0)
    onehot = (batch_ref[...] == gids).astype(jnp.float32)
    pool = jnp.dot(onehot, h, preferred_element_type=jnp.float32)
    q = jnp.maximum(
        jnp.dot(pool, wh1_ref[...], preferred_element_type=jnp.float32)
        + bh1_ref[...], 0.0)
    o_ref[...] = jnp.dot(q, wh2_ref[...],
                         preferred_element_type=jnp.float32) + bh2_ref[...]


_final = pl.pallas_call(
    _final_body,
    out_shape=jax.ShapeDtypeStruct((_G, _OUT), jnp.float32),
)


def kernel(x, edge_index, batch, W1_0, b1_0, W2_0, b2_0, g_0, bt_0,
           W1_1, b1_1, W2_1, b2_1, g_1, bt_1,
           W1_2, b1_2, W2_2, b2_2, g_2, bt_2, Wh1, bh1, Wh2, bh2):
    pad = _EPAD - _E
    src = jnp.concatenate([edge_index[0], jnp.zeros((pad,), jnp.int32)])
    src = src.reshape(_NS, _NCH, _CH)
    dst = jnp.concatenate([edge_index[1], jnp.full((pad,), _NP - 1, jnp.int32)])
    dst = dst.reshape(_NS, _NCH, _CH)
    zrows = jnp.zeros((_CH, _HD), jnp.float32)
    r2 = lambda v: v.reshape(1, -1)
    hl, hh = x[:, :_HD], x[:, _HD:]

    p = _sc_agg(hl, hh, src, dst, zrows)
    hl, hh = _dense(hl, hh, p, W1_0, r2(b1_0), W2_0, r2(b2_0), r2(g_0),
                    r2(bt_0))
    p = _sc_agg(hl, hh, src, dst, zrows)
    hl, hh = _dense(hl, hh, p, W1_1, r2(b1_1), W2_1, r2(b2_1), r2(g_1),
                    r2(bt_1))
    p = _sc_agg(hl, hh, src, dst, zrows)
    return _final(hl, hh, p, batch.reshape(1, -1), W1_2, r2(b1_2), W2_2,
                  r2(b2_2), r2(g_2), r2(bt_2), Wh1, r2(bh1), Wh2, r2(bh2))
